# baseline (speedup 1.0000x reference)
import jax
import jax.numpy as jnp
from jax import lax
from jax.experimental import pallas as pl
from jax.experimental.pallas import tpu as pltpu
from jax.experimental.pallas import tpu_sc as plsc

BATCH = 4096
SEQ = 200
DIM = 64
VOCAB = 1000000
L = 16
NC = 2
NS = 16
NW = NC * NS
BB = BATCH // 128
ST = SEQ // 8
NJOBS = SEQ * BB
JOBS_PER_W = NJOBS // NW
NSLOT = 3
FT = DIM // 8
VPAIR = VOCAB // 2
NVB = (VOCAB + 127) // 128


def _repack_body(tabT_hbm, out1_hbm, in0, in1, ot0, ot1,
                 sem_i0, sem_i1, sem_o0, sem_o1):
    wid = lax.axis_index("s") * NC + lax.axis_index("c")
    in_t = (in0, in1)
    out_t = (ot0, ot1)
    sem_i = (sem_i0, sem_i1)
    sem_o = (sem_o0, sem_o1)
    nvb = lax.select(wid < NVB % NW, NVB // NW + 1, NVB // NW)

    iota = lax.iota(jnp.int32, L)
    c63 = lax.broadcast(jnp.int32(63), (L,))

    def in_desc(i, b):
        vb = wid + i * NW
        return pltpu.make_async_copy(
            tabT_hbm.at[:, pl.ds(vb * 128, 128)], in_t[b], sem_i[b])

    def out_desc_full(i, b):
        vb = wid + i * NW
        return pltpu.make_async_copy(
            out_t[b], out1_hbm.at[pl.ds(vb * 64, 64)], sem_o[b])

    def out_desc_half(i, b):
        vb = wid + i * NW
        return pltpu.make_async_copy(
            out_t[b].at[pl.ds(0, 32)],
            out1_hbm.at[pl.ds(vb * 64, 32)], sem_o[b])

    def start_out(i, b):
        vb = wid + i * NW

        @pl.when(vb < NVB - 1)
        def _():
            out_desc_full(i, b).start()

        @pl.when(vb == NVB - 1)
        def _():
            out_desc_half(i, b).start()

    def wait_out(i, b):
        vb = wid + i * NW

        @pl.when(vb < NVB - 1)
        def _():
            out_desc_full(i, b).wait()

        @pl.when(vb == NVB - 1)
        def _():
            out_desc_half(i, b).wait()

    def transpose(b):
        w16 = [iota + k * L for k in range(DIM // L)]
        rows2c = [[w16[k] * 2 + c for c in range(2)] for k in range(DIM // L)]

        @plsc.parallel_loop(0, DIM, unroll=8)
        def _(f0):
            t = lax.bitwise_and(f0 + iota, c63)
            t64 = t + DIM
            tc = (t, t64)
            for k in range(DIM // L):
                for c in range(2):
                    vals = plsc.load_gather(in_t[b], [t, rows2c[k][c]])
                    plsc.store_scatter(out_t[b], [w16[k], tc[c]], vals)

    @pl.when(nvb > 0)
    def _():
        in_desc(0, 0).start()

    @pl.when(nvb > 1)
    def _():
        in_desc(1, 1).start()

    def body(k, carry):
        for b in range(2):
            i = 2 * k + b

            @pl.when(i < nvb)
            def _():
                in_desc(i, b).wait()

                @pl.when(k > 0)
                def _():
                    wait_out(i - 2, b)

                transpose(b)

                @pl.when(i + 2 < nvb)
                def _():
                    in_desc(i + 2, b).start()

                start_out(i, b)
        return carry

    lax.fori_loop(0, (NVB // NW + 2) // 2, body, 0)

    for b in range(2):
        @pl.when((nvb >= 1) & ((nvb - 1) % 2 == b))
        def _():
            wait_out(nvb - 1, b)

        @pl.when((nvb >= 2) & ((nvb - 2) % 2 == b))
        def _():
            wait_out(nvb - 2, b)


def _emb_body(xp_hbm, table_hbm, out_hbm, idx_v, idx2_v, g0, g1, g2,
              t0, t1, t2, sem_i, sem_g0, sem_g1, sem_g2,
              sem_w0, sem_w1, sem_w2):
    wid = lax.axis_index("s") * NC + lax.axis_index("c")
    job0 = wid * JOBS_PER_W
    gbuf = (g0, g1, g2)
    tbuf = (t0, t1, t2)
    sem_g = (sem_g0, sem_g1, sem_g2)
    sem_w = (sem_w0, sem_w1, sem_w2)

    pltpu.async_copy(xp_hbm.at[pl.ds(job0, JOBS_PER_W)], idx_v, sem_i).wait()

    def halve(r, carry):
        for c in range(128 // L):
            idx2_v[r, pl.ds(c * L, L)] = lax.shift_right_logical(
                idx_v[r, pl.ds(c * L, L)], 1)
        return carry

    lax.fori_loop(0, JOBS_PER_W, halve, 0)

    def gather_desc(jloc, b):
        return pltpu.make_async_copy(
            table_hbm.at[idx2_v.at[jloc]], gbuf[b], sem_g[b])

    def write_desc(jloc, b, ft):
        j = job0 + jloc
        st = j // (BB * 8)
        rem = j % (BB * 8)
        bt = rem // 8
        s = st * 8 + rem % 8
        return pltpu.make_async_copy(
            tbuf[b].at[pl.ds(ft * 8, 8)], out_hbm.at[s, ft, bt], sem_w[b])

    iota = lax.iota(jnp.int32, L)
    c63 = lax.broadcast(jnp.int32(63), (L,))
    rows_c = [iota + c * L for c in range(128 // L)]

    def transpose(jloc, b):
        par64_c = []
        for c in range(128 // L):
            vc = idx_v[jloc, pl.ds(c * L, L)]
            par64_c.append(lax.shift_left(lax.bitwise_and(vc, 1), 6))

        @plsc.parallel_loop(0, DIM, unroll=8)
        def _(f0):
            t = lax.bitwise_and(f0 + iota, c63)
            for c in range(128 // L):
                vals = plsc.load_gather(gbuf[b], [rows_c[c], par64_c[c] + t])
                plsc.store_scatter(tbuf[b], [t, rows_c[c]], vals)

    for b in range(NSLOT):
        gather_desc(b, b).start()

    def body(k, carry):
        for b in range(NSLOT):
            jloc = NSLOT * k + b

            @pl.when(jloc < JOBS_PER_W)
            def _():
                gather_desc(jloc, b).wait()

                @pl.when(k > 0)
                def _():
                    for ft in range(FT):
                        write_desc(jloc - NSLOT, b, ft).wait()

                transpose(jloc, b)

                @pl.when(jloc + NSLOT < JOBS_PER_W)
                def _():
                    gather_desc(jloc + NSLOT, b).start()

                for ft in range(FT):
                    write_desc(jloc, b, ft).start()
        return carry

    lax.fori_loop(0, (JOBS_PER_W + NSLOT - 1) // NSLOT, body, 0)

    for jloc in range(JOBS_PER_W - NSLOT, JOBS_PER_W):
        for ft in range(FT):
            write_desc(jloc, jloc % NSLOT, ft).wait()


def kernel(x, table):
    xp = x.reshape(BB, 128, ST, 8).transpose(2, 0, 3, 1).reshape(NJOBS, 128)
    tabT = table.T
    mesh = plsc.VectorSubcoreMesh(core_axis_name="c", subcore_axis_name="s")
    params = pltpu.CompilerParams(
        use_tc_tiling_on_sc=True, needs_layout_passes=False)

    pairs = pl.kernel(
        _repack_body,
        out_type=jax.ShapeDtypeStruct((VPAIR, 128), jnp.float32),
        mesh=mesh,
        compiler_params=params,
        scratch_types=[
            pltpu.VMEM((DIM, 128), jnp.float32),
            pltpu.VMEM((DIM, 128), jnp.float32),
            pltpu.VMEM((DIM, 128), jnp.float32),
            pltpu.VMEM((DIM, 128), jnp.float32),
            pltpu.SemaphoreType.DMA,
            pltpu.SemaphoreType.DMA,
            pltpu.SemaphoreType.DMA,
            pltpu.SemaphoreType.DMA,
        ],
    )(tabT)

    out = pl.kernel(
        _emb_body,
        out_type=jax.ShapeDtypeStruct((SEQ, FT, BB, 8, 128), jnp.float32),
        mesh=mesh,
        compiler_params=params,
        scratch_types=[
            pltpu.VMEM((JOBS_PER_W, 128), jnp.int32),
            pltpu.VMEM((JOBS_PER_W, 128), jnp.int32),
            pltpu.VMEM((128, 128), jnp.float32),
            pltpu.VMEM((128, 128), jnp.float32),
            pltpu.VMEM((128, 128), jnp.float32),
            pltpu.VMEM((DIM, 128), jnp.float32),
            pltpu.VMEM((DIM, 128), jnp.float32),
            pltpu.VMEM((DIM, 128), jnp.float32),
            pltpu.SemaphoreType.DMA,
            pltpu.SemaphoreType.DMA,
            pltpu.SemaphoreType.DMA,
            pltpu.SemaphoreType.DMA,
            pltpu.SemaphoreType.DMA,
            pltpu.SemaphoreType.DMA,
            pltpu.SemaphoreType.DMA,
        ],
    )(xp, pairs)
    return out.transpose(2, 4, 0, 1, 3).reshape(BATCH, SEQ, DIM)


# speedup vs baseline: 1.0983x; 1.0983x over previous
import jax
import jax.numpy as jnp
from jax import lax
from jax.experimental import pallas as pl
from jax.experimental.pallas import tpu as pltpu
from jax.experimental.pallas import tpu_sc as plsc

BATCH = 4096
SEQ = 200
DIM = 64
VOCAB = 1000000
L = 16
NC = 2
NS = 16
NW = NC * NS
BB = BATCH // 128
ST = SEQ // 8
NJOBS = SEQ * BB
JOBS_PER_W = NJOBS // NW
NSLOT = 3
FT = DIM // 8
VPAIR = VOCAB // 2
NVB = (VOCAB + 127) // 128


def _repack_body(tabT_hbm, out1_hbm, in0, in1, ot0, ot1,
                 sem_i0, sem_i1, sem_o0, sem_o1):
    wid = lax.axis_index("s") * NC + lax.axis_index("c")
    in_t = (in0, in1)
    out_t = (ot0, ot1)
    sem_i = (sem_i0, sem_i1)
    sem_o = (sem_o0, sem_o1)
    nvb = lax.select(wid < NVB % NW, NVB // NW + 1, NVB // NW)

    iota = lax.iota(jnp.int32, L)
    c63 = lax.broadcast(jnp.int32(63), (L,))

    def in_desc(i, b):
        vb = wid + i * NW
        return pltpu.make_async_copy(
            tabT_hbm.at[:, pl.ds(vb * 128, 128)], in_t[b], sem_i[b])

    def out_desc_full(i, b):
        vb = wid + i * NW
        return pltpu.make_async_copy(
            out_t[b], out1_hbm.at[pl.ds(vb * 64, 64)], sem_o[b])

    def out_desc_half(i, b):
        vb = wid + i * NW
        return pltpu.make_async_copy(
            out_t[b].at[pl.ds(0, 32)],
            out1_hbm.at[pl.ds(vb * 64, 32)], sem_o[b])

    def start_out(i, b):
        vb = wid + i * NW

        @pl.when(vb < NVB - 1)
        def _():
            out_desc_full(i, b).start()

        @pl.when(vb == NVB - 1)
        def _():
            out_desc_half(i, b).start()

    def wait_out(i, b):
        vb = wid + i * NW

        @pl.when(vb < NVB - 1)
        def _():
            out_desc_full(i, b).wait()

        @pl.when(vb == NVB - 1)
        def _():
            out_desc_half(i, b).wait()

    def transpose(b):
        w16 = [iota + k * L for k in range(DIM // L)]
        rows2c = [[w16[k] * 2 + c for c in range(2)] for k in range(DIM // L)]

        @plsc.parallel_loop(0, DIM, unroll=2)
        def _(f0):
            t = lax.bitwise_and(f0 + iota, c63)
            t64 = t + DIM
            tc = (t, t64)
            for k in range(DIM // L):
                for c in range(2):
                    vals = plsc.load_gather(in_t[b], [t, rows2c[k][c]])
                    plsc.store_scatter(out_t[b], [w16[k], tc[c]], vals)

    @pl.when(nvb > 0)
    def _():
        in_desc(0, 0).start()

    @pl.when(nvb > 1)
    def _():
        in_desc(1, 1).start()

    def body(k, carry):
        for b in range(2):
            i = 2 * k + b

            @pl.when(i < nvb)
            def _():
                in_desc(i, b).wait()

                @pl.when(k > 0)
                def _():
                    wait_out(i - 2, b)

                transpose(b)

                @pl.when(i + 2 < nvb)
                def _():
                    in_desc(i + 2, b).start()

                start_out(i, b)
        return carry

    lax.fori_loop(0, (NVB // NW + 2) // 2, body, 0)

    for b in range(2):
        @pl.when((nvb >= 1) & ((nvb - 1) % 2 == b))
        def _():
            wait_out(nvb - 1, b)

        @pl.when((nvb >= 2) & ((nvb - 2) % 2 == b))
        def _():
            wait_out(nvb - 2, b)


def _emb_body(xp_hbm, table_hbm, out_hbm, idx_v, idx2_v, g0, g1, g2,
              t0, t1, t2, sem_i, sem_g0, sem_g1, sem_g2,
              sem_w0, sem_w1, sem_w2):
    wid = lax.axis_index("s") * NC + lax.axis_index("c")
    job0 = wid * JOBS_PER_W
    gbuf = (g0, g1, g2)
    tbuf = (t0, t1, t2)
    sem_g = (sem_g0, sem_g1, sem_g2)
    sem_w = (sem_w0, sem_w1, sem_w2)

    pltpu.async_copy(xp_hbm.at[pl.ds(job0, JOBS_PER_W)], idx_v, sem_i).wait()

    def halve(r, carry):
        for c in range(128 // L):
            idx2_v[r, pl.ds(c * L, L)] = lax.shift_right_logical(
                idx_v[r, pl.ds(c * L, L)], 1)
        return carry

    lax.fori_loop(0, JOBS_PER_W, halve, 0)

    def gather_desc(jloc, b):
        return pltpu.make_async_copy(
            table_hbm.at[idx2_v.at[jloc]], gbuf[b], sem_g[b])

    def write_desc(jloc, b, ft):
        j = job0 + jloc
        st = j // (BB * 8)
        rem = j % (BB * 8)
        bt = rem // 8
        s = st * 8 + rem % 8
        return pltpu.make_async_copy(
            tbuf[b].at[pl.ds(ft * 8, 8)], out_hbm.at[s, ft, bt], sem_w[b])

    iota = lax.iota(jnp.int32, L)
    c63 = lax.broadcast(jnp.int32(63), (L,))
    rows_c = [iota + c * L for c in range(128 // L)]

    def transpose(jloc, b):
        par64_c = []
        for c in range(128 // L):
            vc = idx_v[jloc, pl.ds(c * L, L)]
            par64_c.append(lax.shift_left(lax.bitwise_and(vc, 1), 6))

        @plsc.parallel_loop(0, DIM, unroll=2)
        def _(f0):
            t = lax.bitwise_and(f0 + iota, c63)
            for c in range(128 // L):
                vals = plsc.load_gather(gbuf[b], [rows_c[c], par64_c[c] + t])
                plsc.store_scatter(tbuf[b], [t, rows_c[c]], vals)

    for b in range(NSLOT):
        gather_desc(b, b).start()

    def body(k, carry):
        for b in range(NSLOT):
            jloc = NSLOT * k + b

            @pl.when(jloc < JOBS_PER_W)
            def _():
                gather_desc(jloc, b).wait()

                @pl.when(k > 0)
                def _():
                    for ft in range(FT):
                        write_desc(jloc - NSLOT, b, ft).wait()

                transpose(jloc, b)

                @pl.when(jloc + NSLOT < JOBS_PER_W)
                def _():
                    gather_desc(jloc + NSLOT, b).start()

                for ft in range(FT):
                    write_desc(jloc, b, ft).start()
        return carry

    lax.fori_loop(0, (JOBS_PER_W + NSLOT - 1) // NSLOT, body, 0)

    for jloc in range(JOBS_PER_W - NSLOT, JOBS_PER_W):
        for ft in range(FT):
            write_desc(jloc, jloc % NSLOT, ft).wait()


def kernel(x, table):
    xp = x.reshape(BB, 128, ST, 8).transpose(2, 0, 3, 1).reshape(NJOBS, 128)
    tabT = table.T
    mesh = plsc.VectorSubcoreMesh(core_axis_name="c", subcore_axis_name="s")
    params = pltpu.CompilerParams(
        use_tc_tiling_on_sc=True, needs_layout_passes=False)

    pairs = pl.kernel(
        _repack_body,
        out_type=jax.ShapeDtypeStruct((VPAIR, 128), jnp.float32),
        mesh=mesh,
        compiler_params=params,
        scratch_types=[
            pltpu.VMEM((DIM, 128), jnp.float32),
            pltpu.VMEM((DIM, 128), jnp.float32),
            pltpu.VMEM((DIM, 128), jnp.float32),
            pltpu.VMEM((DIM, 128), jnp.float32),
            pltpu.SemaphoreType.DMA,
            pltpu.SemaphoreType.DMA,
            pltpu.SemaphoreType.DMA,
            pltpu.SemaphoreType.DMA,
        ],
    )(tabT)

    out = pl.kernel(
        _emb_body,
        out_type=jax.ShapeDtypeStruct((SEQ, FT, BB, 8, 128), jnp.float32),
        mesh=mesh,
        compiler_params=params,
        scratch_types=[
            pltpu.VMEM((JOBS_PER_W, 128), jnp.int32),
            pltpu.VMEM((JOBS_PER_W, 128), jnp.int32),
            pltpu.VMEM((128, 128), jnp.float32),
            pltpu.VMEM((128, 128), jnp.float32),
            pltpu.VMEM((128, 128), jnp.float32),
            pltpu.VMEM((DIM, 128), jnp.float32),
            pltpu.VMEM((DIM, 128), jnp.float32),
            pltpu.VMEM((DIM, 128), jnp.float32),
            pltpu.SemaphoreType.DMA,
            pltpu.SemaphoreType.DMA,
            pltpu.SemaphoreType.DMA,
            pltpu.SemaphoreType.DMA,
            pltpu.SemaphoreType.DMA,
            pltpu.SemaphoreType.DMA,
            pltpu.SemaphoreType.DMA,
        ],
    )(xp, pairs)
    return out.transpose(2, 4, 0, 1, 3).reshape(BATCH, SEQ, DIM)
